# baseline (speedup 1.0000x reference)
import functools

import jax
import jax.numpy as jnp
from jax import lax
from jax.experimental import pallas as pl
from jax.experimental.pallas import tpu as pltpu
from jax.experimental.pallas import tpu_sc as plsc

B = 16384
D = 64
NC = 2
NS = 16
NW = NC * NS
BW = B // NW
GC = 256

_mesh = plsc.VectorSubcoreMesh(core_axis_name="c", subcore_axis_name="s")

_RL_BLK = 2000


def _tc_relayout_body(x_ref, o_ref):
    hf = pl.program_id(1)

    @pl.when(hf == 0)
    def _():
        o_ref[:, :D] = x_ref[...]

    @pl.when(hf == 1)
    def _():
        o_ref[:, D:] = x_ref[...]


def _tc_relayout(tab):
    n = tab.shape[0]
    h = n // 2
    nb = h // _RL_BLK
    return pl.pallas_call(
        _tc_relayout_body,
        grid=(nb, 2),
        in_specs=[pl.BlockSpec((_RL_BLK, D), lambda b, hf: (b + hf * nb, 0))],
        out_specs=pl.BlockSpec((_RL_BLK, 2 * D), lambda b, hf: (b, 0)),
        out_shape=jax.ShapeDtypeStruct((h, 2 * D), jnp.float32),
        compiler_params=pltpu.CompilerParams(
            dimension_semantics=("parallel", "arbitrary")),
    )(tab)


@functools.partial(
    pl.kernel,
    out_type=(
        jax.ShapeDtypeStruct((B, 2 * D), jnp.float32),
        jax.ShapeDtypeStruct((B, 2 * D), jnp.float32),
        jax.ShapeDtypeStruct((B, 2 * D), jnp.float32),
    ),
    mesh=_mesh,
    scratch_types=[
        pltpu.VMEM((3 * BW,), jnp.int32),
        pltpu.VMEM((2, GC, 2 * D), jnp.float32),
        pltpu.SemaphoreType.DMA,
        pltpu.SemaphoreType.DMA,
        pltpu.SemaphoreType.DMA,
    ],
)
def _sc_gather(u_idx_hbm, p_idx_hbm, n_idx_hbm, user_p_hbm, item_p_hbm,
               u_out, p_out, n_out, tix_v, pairs_v, isem, gsem, wsem):
    wid = lax.axis_index("s") * NC + lax.axis_index("c")
    base = wid * BW
    sl = pl.ds(base, BW)

    pltpu.async_copy(u_idx_hbm.at[sl], tix_v.at[pl.ds(0, BW)], isem)
    pltpu.async_copy(p_idx_hbm.at[sl], tix_v.at[pl.ds(BW, BW)], isem)
    pltpu.async_copy(n_idx_hbm.at[sl], tix_v.at[pl.ds(2 * BW, BW)], isem)
    for _ in range(3):
        pltpu.make_async_copy(
            u_idx_hbm.at[sl], tix_v.at[pl.ds(0, BW)], isem).wait()

    @pl.loop(0, 3 * BW, step=16)
    def _(k):
        v = tix_v[pl.ds(k, 16)]
        half_n = jnp.where(k < BW, 50000, 500000)
        tix_v[pl.ds(k, 16)] = jnp.where(v >= half_n, v - half_n, v)

    tabs = (user_p_hbm, item_p_hbm, item_p_hbm)
    outs = (u_out, p_out, n_out)
    nch = BW // GC
    total = 3 * nch

    def fire(g, buf):
        t, c = g // nch, g % nch
        idx = tix_v.at[pl.ds(t * BW + c * GC, GC)]
        pltpu.async_copy(tabs[t].at[idx], pairs_v.at[buf], gsem)

    def wait_gather(buf):
        pltpu.make_async_copy(
            user_p_hbm.at[tix_v.at[pl.ds(0, GC)]], pairs_v.at[buf], gsem
        ).wait()

    def writeback(g, buf):
        t, c = g // nch, g % nch
        pltpu.async_copy(
            pairs_v.at[buf], outs[t].at[pl.ds(base + c * GC, GC)], wsem)

    fire(0, 0)
    for g in range(total):
        buf = g % 2
        wait_gather(buf)
        if g >= 1:
            pltpu.make_async_copy(
                pairs_v.at[0], u_out.at[pl.ds(0, GC)], wsem).wait()
        if g + 1 < total:
            fire(g + 1, (g + 1) % 2)
        writeback(g, buf)
    pltpu.make_async_copy(
        pairs_v.at[0], u_out.at[pl.ds(0, GC)], wsem).wait()


_TC_BLK = 2048


def _tc_dist_body(bu_ref, bp_ref, bn_ref, u_ref, i_ref, j_ref,
                  pos_ref, neg_ref):
    def pick(pair_ref, idx_ref, half_n):
        x = pair_ref[...]
        hi = idx_ref[...] >= half_n
        return jnp.where(hi, x[:, D:], x[:, :D])

    u = pick(u_ref, bu_ref, 50000)
    i = pick(i_ref, bp_ref, 500000)
    j = pick(j_ref, bn_ref, 500000)
    ssq_u = jnp.sum(u * u, axis=1, keepdims=True)
    ssq_i = jnp.sum(i * i, axis=1, keepdims=True)
    ssq_j = jnp.sum(j * j, axis=1, keepdims=True)
    dot_i = jnp.sum(u * i, axis=1, keepdims=True)
    dot_j = jnp.sum(u * j, axis=1, keepdims=True)
    mu = jnp.maximum(ssq_u, 1.0)
    mi = jnp.maximum(ssq_i, 1.0)
    mj = jnp.maximum(ssq_j, 1.0)
    pos_ref[...] = ssq_u / mu + ssq_i / mi - 2.0 * dot_i * lax.rsqrt(mu * mi)
    neg_ref[...] = ssq_u / mu + ssq_j / mj - 2.0 * dot_j * lax.rsqrt(mu * mj)


def _tc_dist(bu, bp, bn, u_pairs, p_pairs, n_pairs):
    pair_spec = pl.BlockSpec((_TC_BLK, 2 * D), lambda b: (b, 0))
    idx_spec = pl.BlockSpec((_TC_BLK, 1), lambda b: (b, 0))
    out_spec = pl.BlockSpec((_TC_BLK, 1), lambda b: (b, 0))
    return pl.pallas_call(
        _tc_dist_body,
        grid=(B // _TC_BLK,),
        in_specs=[idx_spec, idx_spec, idx_spec, pair_spec, pair_spec, pair_spec],
        out_specs=[out_spec, out_spec],
        out_shape=[
            jax.ShapeDtypeStruct((B, 1), jnp.float32),
            jax.ShapeDtypeStruct((B, 1), jnp.float32),
        ],
        compiler_params=pltpu.CompilerParams(
            dimension_semantics=("parallel",)),
    )(bu, bp, bn, u_pairs, p_pairs, n_pairs)


def kernel(batch_user, batch_pos_item, batch_neg_item, user_emb, item_emb):
    user_p = _tc_relayout(user_emb)
    item_p = _tc_relayout(item_emb)
    u_pairs, p_pairs, n_pairs = _sc_gather(
        batch_user, batch_pos_item, batch_neg_item, user_p, item_p)
    pos, neg = _tc_dist(
        batch_user.reshape(B, 1), batch_pos_item.reshape(B, 1),
        batch_neg_item.reshape(B, 1), u_pairs, p_pairs, n_pairs)
    return (pos, neg)


# speedup vs baseline: 1.4490x; 1.4490x over previous
import functools

import jax
import jax.numpy as jnp
from jax import lax
from jax.experimental import pallas as pl
from jax.experimental.pallas import tpu as pltpu
from jax.experimental.pallas import tpu_sc as plsc

B = 16384
D = 64
NC = 2
NS = 16
NW = NC * NS
BW = B // NW

_mesh = plsc.VectorSubcoreMesh(core_axis_name="c", subcore_axis_name="s")


@functools.partial(
    pl.kernel,
    out_type=(
        jax.ShapeDtypeStruct((B, D), jnp.float32),
        jax.ShapeDtypeStruct((B, D), jnp.float32),
        jax.ShapeDtypeStruct((B, D), jnp.float32),
    ),
    mesh=_mesh,
    compiler_params=pltpu.CompilerParams(use_tc_tiling_on_sc=False),
    scratch_types=[
        pltpu.VMEM((BW,), jnp.int32),
        pltpu.VMEM((BW,), jnp.int32),
        pltpu.VMEM((BW,), jnp.int32),
        pltpu.VMEM((BW, D), jnp.float32),
        pltpu.VMEM((BW, D), jnp.float32),
        pltpu.VMEM((BW, D), jnp.float32),
        pltpu.SemaphoreType.DMA,
        pltpu.SemaphoreType.DMA,
    ],
)
def _sc_gather(u_idx_hbm, p_idx_hbm, n_idx_hbm, user_hbm, item_hbm,
               u_out, p_out, n_out,
               iu_v, ip_v, in_v, ru_v, rp_v, rn_v, gsem, wsem):
    wid = lax.axis_index("s") * NC + lax.axis_index("c")
    base = wid * BW
    sl = pl.ds(base, BW)
    pltpu.sync_copy(u_idx_hbm.at[sl], iu_v)
    pltpu.sync_copy(p_idx_hbm.at[sl], ip_v)
    pltpu.sync_copy(n_idx_hbm.at[sl], in_v)
    g0 = pltpu.async_copy(user_hbm.at[iu_v], ru_v, gsem)
    g1 = pltpu.async_copy(item_hbm.at[ip_v], rp_v, gsem)
    g2 = pltpu.async_copy(item_hbm.at[in_v], rn_v, gsem)
    g0.wait()
    w0 = pltpu.async_copy(ru_v, u_out.at[sl], wsem)
    g1.wait()
    w1 = pltpu.async_copy(rp_v, p_out.at[sl], wsem)
    g2.wait()
    w2 = pltpu.async_copy(rn_v, n_out.at[sl], wsem)
    w0.wait()
    w1.wait()
    w2.wait()


_TC_BLK = 2048


def _tc_dist_body(u_ref, i_ref, j_ref, pos_ref, neg_ref):
    u = u_ref[...]
    i = i_ref[...]
    j = j_ref[...]
    ssq_u = jnp.sum(u * u, axis=1, keepdims=True)
    ssq_i = jnp.sum(i * i, axis=1, keepdims=True)
    ssq_j = jnp.sum(j * j, axis=1, keepdims=True)
    dot_i = jnp.sum(u * i, axis=1, keepdims=True)
    dot_j = jnp.sum(u * j, axis=1, keepdims=True)
    mu = jnp.maximum(ssq_u, 1.0)
    mi = jnp.maximum(ssq_i, 1.0)
    mj = jnp.maximum(ssq_j, 1.0)
    pos_ref[...] = ssq_u / mu + ssq_i / mi - 2.0 * dot_i * lax.rsqrt(mu * mi)
    neg_ref[...] = ssq_u / mu + ssq_j / mj - 2.0 * dot_j * lax.rsqrt(mu * mj)


def _tc_dist(u_rows, p_rows, n_rows):
    row_spec = pl.BlockSpec((_TC_BLK, D), lambda b: (b, 0))
    out_spec = pl.BlockSpec((_TC_BLK, 1), lambda b: (b, 0))
    return pl.pallas_call(
        _tc_dist_body,
        grid=(B // _TC_BLK,),
        in_specs=[row_spec, row_spec, row_spec],
        out_specs=[out_spec, out_spec],
        out_shape=[
            jax.ShapeDtypeStruct((B, 1), jnp.float32),
            jax.ShapeDtypeStruct((B, 1), jnp.float32),
        ],
    )(u_rows, p_rows, n_rows)


def kernel(batch_user, batch_pos_item, batch_neg_item, user_emb, item_emb):
    u_rows, p_rows, n_rows = _sc_gather(
        batch_user, batch_pos_item, batch_neg_item, user_emb, item_emb)
    pos, neg = _tc_dist(u_rows, p_rows, n_rows)
    return (pos, neg)
